# Initial kernel scaffold; baseline (speedup 1.0000x reference)
import jax
import jax.numpy as jnp
from jax.experimental import pallas as pl


def kernel(drug_emb, dis_emb, r_d_edge, train_index, train):
    raise NotImplementedError("write your pallas kernel here")


# speedup vs baseline: 2.8645x; 2.8645x over previous
import functools

import jax
import jax.numpy as jnp
from jax import lax
from jax.experimental import pallas as pl
from jax.experimental.pallas import tpu as pltpu
from jax.experimental.pallas import tpu_sc as plsc

D = 32
CH = 128
L = 16


def _build_sc_kernel(n_pad, n_workers, per_w):
    nchunk = per_w // CH
    mesh = plsc.VectorSubcoreMesh(core_axis_name="c", subcore_axis_name="s")

    @functools.partial(
        pl.kernel,
        out_type=jax.ShapeDtypeStruct((n_pad,), jnp.float32),
        mesh=mesh,
        compiler_params=pltpu.CompilerParams(
            needs_layout_passes=False, use_tc_tiling_on_sc=False),
        scratch_types=[
            pltpu.VMEM((CH,), jnp.int32),
            pltpu.VMEM((CH,), jnp.int32),
            pltpu.VMEM((CH,), jnp.int32),
            pltpu.VMEM((CH, D), jnp.float32),
            pltpu.VMEM((CH, D), jnp.float32),
            pltpu.VMEM((CH,), jnp.float32),
            pltpu.SemaphoreType.DMA,
            pltpu.SemaphoreType.DMA,
        ],
    )
    def sc_kernel(row_hbm, col_hbm, drug_hbm, dis_hbm, idx_hbm, out_hbm,
                  idx_v, row_v, col_v, d_v, s_v, out_v, sem1, sem2):
        n_cores = 2
        wid = lax.axis_index("s") * n_cores + lax.axis_index("c")
        base = wid * per_w
        lanes = lax.iota(jnp.int32, L)

        def chunk_body(ci, carry):
            off = base + ci * CH
            pltpu.sync_copy(idx_hbm.at[pl.ds(off, CH)], idx_v)
            cr = pltpu.async_copy(row_hbm.at[idx_v], row_v, sem1)
            cc = pltpu.async_copy(col_hbm.at[idx_v], col_v, sem2)
            cr.wait()
            cc.wait()

            cd = pltpu.async_copy(drug_hbm.at[row_v], d_v, sem1)
            cs = pltpu.async_copy(dis_hbm.at[col_v], s_v, sem2)
            cd.wait()
            cs.wait()

            def dot_body(g, c):
                r16 = g * L + lanes
                acc = jnp.zeros((L,), jnp.float32)
                for j in range(D):
                    jv = jnp.full((L,), j, jnp.int32)
                    acc = acc + (plsc.load_gather(d_v, [r16, jv]) *
                                 plsc.load_gather(s_v, [r16, jv]))
                out_v[pl.ds(g * L, L)] = 1.0 / (1.0 + jnp.exp(-acc))
                return c

            lax.fori_loop(0, CH // L, dot_body, 0)
            pltpu.sync_copy(out_v, out_hbm.at[pl.ds(off, CH)])
            return carry

        lax.fori_loop(0, nchunk, chunk_body, 0)

    return sc_kernel


def kernel(drug_emb, dis_emb, r_d_edge, train_index, train):
    n = train_index.shape[0]
    n_workers = 32
    nchunk = -(-n // (n_workers * CH))
    per_w = nchunk * CH
    n_pad = n_workers * per_w

    edge32 = r_d_edge.astype(jnp.int32)
    row_arr = edge32[0]
    col_arr = edge32[1]
    idx = train_index.astype(jnp.int32)
    idx_p = jnp.concatenate([idx, jnp.zeros((n_pad - n,), jnp.int32)])

    sc = _build_sc_kernel(n_pad, n_workers, per_w)
    out = sc(row_arr, col_arr, drug_emb, dis_emb, idx_p)
    return out[:n]
